# baseline (speedup 1.0000x reference)
import functools

import jax
import jax.numpy as jnp
from jax import lax
from jax.experimental import pallas as pl
from jax.experimental.pallas import tpu as pltpu
from jax.experimental.pallas import tpu_sc as plsc

N = 10000
E = 320000
D = 128

NC = 2
NS = 16
NW = NC * NS
CHUNK = 128
CD = -(-E // (NW * CHUNK))
C1 = 158
C1S = C1 + 1
EP = NW * CD * CHUNK
EP1 = NS * C1 * CHUNK
NP = 10240
RPT = NP // NS

_mesh = plsc.VectorSubcoreMesh(core_axis_name="c", subcore_axis_name="s")


@functools.partial(
    pl.kernel,
    out_type=(
        jax.ShapeDtypeStruct((NC, NP), jnp.float32),
        jax.ShapeDtypeStruct((NC, NP), jnp.float32),
    ),
    mesh=_mesh,
    scratch_types=[
        pltpu.VMEM((CHUNK,), jnp.int32),
        pltpu.VMEM((CHUNK,), jnp.int32),
        pltpu.VMEM((CHUNK,), jnp.float32),
        pltpu.VMEM_SHARED((NP,), jnp.float32),
        pltpu.VMEM_SHARED((NP,), jnp.float32),
    ],
)
def _sc_degrees(src_hbm, dst_hbm, zeros_hbm, do_hbm, di_hbm,
                sidx, didx, ones_v, acc_o, acc_i):
    cid = lax.axis_index("c")
    sid = lax.axis_index("s")
    wid = cid * NS + sid
    for i in range(CHUNK // 16):
        ones_v[pl.ds(i * 16, 16)] = jnp.ones((16,), jnp.float32)
    base = sid * RPT
    for k in range(RPT // CHUNK):
        pltpu.sync_copy(zeros_hbm.at[k], acc_o.at[pl.ds(base + k * CHUNK, CHUNK)])
        pltpu.sync_copy(zeros_hbm.at[k], acc_i.at[pl.ds(base + k * CHUNK, CHUNK)])
    plsc.subcore_barrier()

    def body(j, _):
        pltpu.sync_copy(src_hbm.at[wid, j], sidx)
        pltpu.sync_copy(dst_hbm.at[wid, j], didx)
        pltpu.sync_copy(ones_v, acc_o.at[sidx], add=True)
        pltpu.sync_copy(ones_v, acc_i.at[didx], add=True)
        return 0

    lax.fori_loop(0, CD, body, 0)
    plsc.subcore_barrier()
    pltpu.sync_copy(acc_o.at[pl.ds(base, RPT)], do_hbm.at[cid, pl.ds(base, RPT)])
    pltpu.sync_copy(acc_i.at[pl.ds(base, RPT)], di_hbm.at[cid, pl.ds(base, RPT)])


@functools.partial(
    pl.kernel,
    out_type=jax.ShapeDtypeStruct((NP, D), jnp.float32),
    mesh=_mesh,
    scratch_types=[
        pltpu.VMEM((CHUNK,), jnp.int32),
        pltpu.VMEM((CHUNK,), jnp.int32),
        pltpu.VMEM((CHUNK,), jnp.int32),
        pltpu.VMEM((CHUNK,), jnp.int32),
        pltpu.VMEM((CHUNK, D), jnp.float32),
        pltpu.VMEM((CHUNK, D), jnp.float32),
        pltpu.VMEM_SHARED((NP, D), jnp.float32),
        pltpu.SemaphoreType.DMA,
        pltpu.SemaphoreType.DMA,
    ],
)
def _sc_aggregate(hs_hbm, src_hbm, dst_hbm, zeros_hbm, out_hbm,
                  sidx_a, sidx_b, didx_a, didx_b, rows_a, rows_b,
                  acc, sem_a, sem_b):
    cid = lax.axis_index("c")
    sid = lax.axis_index("s")
    base = sid * RPT

    @pl.when(cid == 0)
    def _():
        pltpu.sync_copy(zeros_hbm.at[pl.ds(base, RPT)], acc.at[pl.ds(base, RPT)])
        plsc.subcore_barrier()
        pltpu.sync_copy(src_hbm.at[sid, 0], sidx_a)
        pltpu.async_copy(hs_hbm.at[sidx_a], rows_a, sem_a)

        def body(g, _):
            c0 = 2 * g
            pltpu.sync_copy(src_hbm.at[sid, c0 + 1], sidx_b)
            pltpu.async_copy(hs_hbm.at[sidx_b], rows_b, sem_b)
            pltpu.sync_copy(dst_hbm.at[sid, c0], didx_a)
            pltpu.make_async_copy(hs_hbm.at[sidx_a], rows_a, sem_a).wait()
            pltpu.sync_copy(rows_a, acc.at[didx_a], add=True)
            pltpu.sync_copy(src_hbm.at[sid, c0 + 2], sidx_a)
            pltpu.async_copy(hs_hbm.at[sidx_a], rows_a, sem_a)
            pltpu.sync_copy(dst_hbm.at[sid, c0 + 1], didx_b)
            pltpu.make_async_copy(hs_hbm.at[sidx_b], rows_b, sem_b).wait()
            pltpu.sync_copy(rows_b, acc.at[didx_b], add=True)
            return 0

        lax.fori_loop(0, C1 // 2, body, 0)
        pltpu.make_async_copy(hs_hbm.at[sidx_a], rows_a, sem_a).wait()
        plsc.subcore_barrier()
        pltpu.sync_copy(acc.at[pl.ds(base, RPT)], out_hbm.at[pl.ds(base, RPT)])


_R = NP // 8


def _tc_first_body(dop, dip, x, w, hs, ns, nd):
    n_s = lax.rsqrt(dop[0] + dop[1] + 1.0)
    n_d = lax.rsqrt(dip[0] + dip[1] + 1.0)
    ns[...] = n_s
    nd[...] = n_d
    hs[...] = jnp.dot(x[...] * n_s, w[...], preferred_element_type=jnp.float32)


def _tc_first(dop, dip, x, w):
    return pl.pallas_call(
        _tc_first_body,
        grid=(8,),
        in_specs=[
            pl.BlockSpec((NC, _R, 1), lambda i: (0, i, 0)),
            pl.BlockSpec((NC, _R, 1), lambda i: (0, i, 0)),
            pl.BlockSpec((_R, D), lambda i: (i, 0)),
            pl.BlockSpec((D, D), lambda i: (0, 0)),
        ],
        out_specs=[
            pl.BlockSpec((_R, D), lambda i: (i, 0)),
            pl.BlockSpec((_R, 1), lambda i: (i, 0)),
            pl.BlockSpec((_R, 1), lambda i: (i, 0)),
        ],
        out_shape=[
            jax.ShapeDtypeStruct((NP, D), jnp.float32),
            jax.ShapeDtypeStruct((NP, 1), jnp.float32),
            jax.ShapeDtypeStruct((NP, 1), jnp.float32),
        ],
    )(dop, dip, x, w)


def _tc_mid_body(p, hs, nd, ns, b, w, out):
    h = jnp.maximum((p[...] + hs[...]) * nd[...] + b[...], 0.0)
    out[...] = jnp.dot(h * ns[...], w[...], preferred_element_type=jnp.float32)


def _tc_mid(p, hs, nd, ns, b, w):
    return pl.pallas_call(
        _tc_mid_body,
        grid=(8,),
        in_specs=[
            pl.BlockSpec((_R, D), lambda i: (i, 0)),
            pl.BlockSpec((_R, D), lambda i: (i, 0)),
            pl.BlockSpec((_R, 1), lambda i: (i, 0)),
            pl.BlockSpec((_R, 1), lambda i: (i, 0)),
            pl.BlockSpec((1, D), lambda i: (0, 0)),
            pl.BlockSpec((D, D), lambda i: (0, 0)),
        ],
        out_specs=pl.BlockSpec((_R, D), lambda i: (i, 0)),
        out_shape=jax.ShapeDtypeStruct((NP, D), jnp.float32),
    )(p, hs, nd, ns, b, w)


def _tc_last_body(p, hs, nd, b, out):
    out[...] = (p[...] + hs[...]) * nd[...] + b[...]


def _tc_last(p, hs, nd, b):
    return pl.pallas_call(
        _tc_last_body,
        grid=(8,),
        in_specs=[
            pl.BlockSpec((_R, D), lambda i: (i, 0)),
            pl.BlockSpec((_R, D), lambda i: (i, 0)),
            pl.BlockSpec((_R, 1), lambda i: (i, 0)),
            pl.BlockSpec((1, D), lambda i: (0, 0)),
        ],
        out_specs=pl.BlockSpec((_R, D), lambda i: (i, 0)),
        out_shape=jax.ShapeDtypeStruct((NP, D), jnp.float32),
    )(p, hs, nd, b)


def kernel(features, edge_index, W1, b1, W2, b2, W3, b3):
    src = edge_index[0].astype(jnp.int32)
    dst = edge_index[1].astype(jnp.int32)
    srcd = jnp.pad(src, (0, EP - E), constant_values=N).reshape(NW, CD, CHUNK)
    dstd = jnp.pad(dst, (0, EP - E), constant_values=N).reshape(NW, CD, CHUNK)
    padc = jnp.full((NS, 1, CHUNK), N, jnp.int32)
    src1 = jnp.concatenate(
        [jnp.pad(src, (0, EP1 - E), constant_values=N).reshape(NS, C1, CHUNK),
         padc], axis=1)
    dst1 = jnp.concatenate(
        [jnp.pad(dst, (0, EP1 - E), constant_values=N).reshape(NS, C1, CHUNK),
         padc], axis=1)
    xp = jnp.pad(features, ((0, NP - N), (0, 0)))
    zeros = jnp.zeros((NP, D), jnp.float32)

    do_p, di_p = _sc_degrees(srcd, dstd, zeros)
    dop = do_p.reshape(NC, NP, 1)
    dip = di_p.reshape(NC, NP, 1)

    hs, ns, nd = _tc_first(dop, dip, xp, W1)
    b1r = b1.reshape(1, D)
    b2r = b2.reshape(1, D)
    b3r = b3.reshape(1, D)

    p = _sc_aggregate(hs, src1, dst1, zeros)
    hs = _tc_mid(p, hs, nd, ns, b1r, W2)
    p = _sc_aggregate(hs, src1, dst1, zeros)
    hs = _tc_mid(p, hs, nd, ns, b2r, W2)
    p = _sc_aggregate(hs, src1, dst1, zeros)
    hs = _tc_mid(p, hs, nd, ns, b2r, W3)
    p = _sc_aggregate(hs, src1, dst1, zeros)
    out = _tc_last(p, hs, nd, b3r)
    return out[:N]


# speedup vs baseline: 1.1315x; 1.1315x over previous
import functools

import jax
import jax.numpy as jnp
from jax import lax
from jax.experimental import pallas as pl
from jax.experimental.pallas import tpu as pltpu
from jax.experimental.pallas import tpu_sc as plsc

N = 10000
E = 320000
D = 128

NC = 2
NS = 16
NW = NC * NS
CHUNK = 128
C = -(-E // (NW * CHUNK))
EP = NW * C * CHUNK
NP = 10240
RPT = NP // NS

_mesh = plsc.VectorSubcoreMesh(core_axis_name="c", subcore_axis_name="s")


@functools.partial(
    pl.kernel,
    out_type=(
        jax.ShapeDtypeStruct((NC, NP), jnp.float32),
        jax.ShapeDtypeStruct((NC, NP), jnp.float32),
    ),
    mesh=_mesh,
    scratch_types=[
        pltpu.VMEM((CHUNK,), jnp.int32),
        pltpu.VMEM((CHUNK,), jnp.int32),
        pltpu.VMEM((CHUNK,), jnp.float32),
        pltpu.VMEM_SHARED((NP,), jnp.float32),
        pltpu.VMEM_SHARED((NP,), jnp.float32),
    ],
)
def _sc_degrees(src_hbm, dst_hbm, zeros_hbm, do_hbm, di_hbm,
                sidx, didx, ones_v, acc_o, acc_i):
    cid = lax.axis_index("c")
    sid = lax.axis_index("s")
    wid = cid * NS + sid
    for i in range(CHUNK // 16):
        ones_v[pl.ds(i * 16, 16)] = jnp.ones((16,), jnp.float32)
    base = sid * RPT
    for k in range(RPT // CHUNK):
        pltpu.sync_copy(zeros_hbm.at[k], acc_o.at[pl.ds(base + k * CHUNK, CHUNK)])
        pltpu.sync_copy(zeros_hbm.at[k], acc_i.at[pl.ds(base + k * CHUNK, CHUNK)])
    plsc.subcore_barrier()

    def body(j, _):
        pltpu.sync_copy(src_hbm.at[wid, j], sidx)
        pltpu.sync_copy(dst_hbm.at[wid, j], didx)
        pltpu.sync_copy(ones_v, acc_o.at[sidx], add=True)
        pltpu.sync_copy(ones_v, acc_i.at[didx], add=True)
        return 0

    lax.fori_loop(0, C, body, 0)
    plsc.subcore_barrier()
    pltpu.sync_copy(acc_o.at[pl.ds(base, RPT)], do_hbm.at[cid, pl.ds(base, RPT)])
    pltpu.sync_copy(acc_i.at[pl.ds(base, RPT)], di_hbm.at[cid, pl.ds(base, RPT)])


@functools.partial(
    pl.kernel,
    out_type=jax.ShapeDtypeStruct((NC, NP, D), jnp.float32),
    mesh=_mesh,
    scratch_types=[
        pltpu.VMEM((CHUNK,), jnp.int32),
        pltpu.VMEM((CHUNK,), jnp.int32),
        pltpu.VMEM((CHUNK, D), jnp.float32),
        pltpu.VMEM_SHARED((NP, D), jnp.float32),
        pltpu.SemaphoreType.DMA,
    ],
)
def _sc_aggregate(hs_hbm, src_hbm, dst_hbm, zeros_hbm, out_hbm,
                  sidx, didx, rows, acc, sem):
    cid = lax.axis_index("c")
    sid = lax.axis_index("s")
    wid = cid * NS + sid
    base = sid * RPT
    pltpu.sync_copy(zeros_hbm.at[pl.ds(base, RPT)], acc.at[pl.ds(base, RPT)])
    plsc.subcore_barrier()

    def body(j, _):
        pltpu.sync_copy(src_hbm.at[wid, j], sidx)
        pltpu.async_copy(hs_hbm.at[sidx], rows, sem).wait()
        pltpu.sync_copy(dst_hbm.at[wid, j], didx)
        pltpu.sync_copy(rows, acc.at[didx], add=True)
        return 0

    lax.fori_loop(0, C, body, 0)
    plsc.subcore_barrier()
    pltpu.sync_copy(acc.at[pl.ds(base, RPT)], out_hbm.at[cid, pl.ds(base, RPT)])


_R = NP // 8


def _tc_first_body(dop, dip, x, w, hs, ns, nd):
    n_s = lax.rsqrt(dop[0] + dop[1] + 1.0)
    n_d = lax.rsqrt(dip[0] + dip[1] + 1.0)
    ns[...] = n_s
    nd[...] = n_d
    hs[...] = jnp.dot(x[...] * n_s, w[...], preferred_element_type=jnp.float32)


def _tc_first(dop, dip, x, w):
    return pl.pallas_call(
        _tc_first_body,
        grid=(8,),
        in_specs=[
            pl.BlockSpec((NC, _R, 1), lambda i: (0, i, 0)),
            pl.BlockSpec((NC, _R, 1), lambda i: (0, i, 0)),
            pl.BlockSpec((_R, D), lambda i: (i, 0)),
            pl.BlockSpec((D, D), lambda i: (0, 0)),
        ],
        out_specs=[
            pl.BlockSpec((_R, D), lambda i: (i, 0)),
            pl.BlockSpec((_R, 1), lambda i: (i, 0)),
            pl.BlockSpec((_R, 1), lambda i: (i, 0)),
        ],
        out_shape=[
            jax.ShapeDtypeStruct((NP, D), jnp.float32),
            jax.ShapeDtypeStruct((NP, 1), jnp.float32),
            jax.ShapeDtypeStruct((NP, 1), jnp.float32),
        ],
    )(dop, dip, x, w)


def _tc_mid_body(p, hs, nd, ns, b, w, out):
    agg = p[0] + p[1] + hs[...]
    h = jnp.maximum(agg * nd[...] + b[...], 0.0)
    out[...] = jnp.dot(h * ns[...], w[...], preferred_element_type=jnp.float32)


def _tc_mid(p, hs, nd, ns, b, w):
    return pl.pallas_call(
        _tc_mid_body,
        grid=(8,),
        in_specs=[
            pl.BlockSpec((NC, _R, D), lambda i: (0, i, 0)),
            pl.BlockSpec((_R, D), lambda i: (i, 0)),
            pl.BlockSpec((_R, 1), lambda i: (i, 0)),
            pl.BlockSpec((_R, 1), lambda i: (i, 0)),
            pl.BlockSpec((1, D), lambda i: (0, 0)),
            pl.BlockSpec((D, D), lambda i: (0, 0)),
        ],
        out_specs=pl.BlockSpec((_R, D), lambda i: (i, 0)),
        out_shape=jax.ShapeDtypeStruct((NP, D), jnp.float32),
    )(p, hs, nd, ns, b, w)


def _tc_last_body(p, hs, nd, b, out):
    out[...] = (p[0] + p[1] + hs[...]) * nd[...] + b[...]


def _tc_last(p, hs, nd, b):
    return pl.pallas_call(
        _tc_last_body,
        grid=(8,),
        in_specs=[
            pl.BlockSpec((NC, _R, D), lambda i: (0, i, 0)),
            pl.BlockSpec((_R, D), lambda i: (i, 0)),
            pl.BlockSpec((_R, 1), lambda i: (i, 0)),
            pl.BlockSpec((1, D), lambda i: (0, 0)),
        ],
        out_specs=pl.BlockSpec((_R, D), lambda i: (i, 0)),
        out_shape=jax.ShapeDtypeStruct((NP, D), jnp.float32),
    )(p, hs, nd, b)


def kernel(features, edge_index, W1, b1, W2, b2, W3, b3):
    src = edge_index[0].astype(jnp.int32)
    dst = edge_index[1].astype(jnp.int32)
    src3 = jnp.pad(src, (0, EP - E), constant_values=N).reshape(NW, C, CHUNK)
    dst3 = jnp.pad(dst, (0, EP - E), constant_values=N).reshape(NW, C, CHUNK)
    xp = jnp.pad(features, ((0, NP - N), (0, 0)))
    zeros = jnp.zeros((NP, D), jnp.float32)

    do_p, di_p = _sc_degrees(src3, dst3, zeros)
    dop = do_p.reshape(NC, NP, 1)
    dip = di_p.reshape(NC, NP, 1)

    hs, ns, nd = _tc_first(dop, dip, xp, W1)
    b1r = b1.reshape(1, D)
    b2r = b2.reshape(1, D)
    b3r = b3.reshape(1, D)

    p = _sc_aggregate(hs, src3, dst3, zeros)
    hs = _tc_mid(p, hs, nd, ns, b1r, W2)
    p = _sc_aggregate(hs, src3, dst3, zeros)
    hs = _tc_mid(p, hs, nd, ns, b2r, W2)
    p = _sc_aggregate(hs, src3, dst3, zeros)
    hs = _tc_mid(p, hs, nd, ns, b2r, W3)
    p = _sc_aggregate(hs, src3, dst3, zeros)
    out = _tc_last(p, hs, nd, b3r)
    return out[:N]
